# baseline (speedup 1.0000x reference)
import jax
import jax.numpy as jnp
from jax.experimental import pallas as pl


def _pos_kernel(col_ref, row_ref, o_ref):
    nc = col_ref.shape[1]
    h = row_ref.shape[0]
    w = col_ref.shape[0]
    col_t = col_ref[...].T
    row_t = row_ref[...].T
    col_tile = jnp.broadcast_to(col_t[:, None, :], (nc, h, w))
    row_tile = jnp.broadcast_to(row_t[:, :, None], (nc, h, w))
    o_ref[0, :nc] = col_tile
    o_ref[0, nc:] = row_tile


def kernel(x, row_w, col_w):
    b = x.shape[0]
    h, w = x.shape[-2], x.shape[-1]
    nc = row_w.shape[1]
    out_shape = jax.ShapeDtypeStruct((b, 2 * nc, h, w), jnp.float32)
    return pl.pallas_call(
        _pos_kernel,
        grid=(b,),
        in_specs=[
            pl.BlockSpec((w, nc), lambda i: (0, 0)),
            pl.BlockSpec((h, nc), lambda i: (0, 0)),
        ],
        out_specs=pl.BlockSpec((1, 2 * nc, h, w), lambda i: (i, 0, 0, 0)),
        out_shape=out_shape,
    )(col_w, row_w)


# speedup vs baseline: 3.1468x; 3.1468x over previous
import jax
import jax.numpy as jnp
from jax.experimental import pallas as pl


def _pos_kernel(col_ref, row_ref, o_ref):
    bb = o_ref.shape[0]
    nc = col_ref.shape[1]
    h = row_ref.shape[0]
    w = col_ref.shape[0]
    col_t = col_ref[...].T
    row_t = row_ref[...].T
    col_part = jnp.broadcast_to(col_t[:, None, :], (nc, h, w)).reshape(nc, h * w)
    row_part = jnp.broadcast_to(row_t[:, :, None], (nc, h, w)).reshape(nc, h * w)
    tile = jnp.concatenate([col_part, row_part], axis=0)
    o_ref[...] = jnp.broadcast_to(tile[None], (bb, 2 * nc, h * w))


def kernel(x, row_w, col_w):
    b = x.shape[0]
    h, w = x.shape[-2], x.shape[-1]
    nc = row_w.shape[1]
    bb = 8
    out_shape = jax.ShapeDtypeStruct((b, 2 * nc, h * w), jnp.float32)
    out = pl.pallas_call(
        _pos_kernel,
        grid=(b // bb,),
        in_specs=[
            pl.BlockSpec((w, nc), lambda i: (0, 0)),
            pl.BlockSpec((h, nc), lambda i: (0, 0)),
        ],
        out_specs=pl.BlockSpec((bb, 2 * nc, h * w), lambda i: (i, 0, 0)),
        out_shape=out_shape,
    )(col_w, row_w)
    return out.reshape(b, 2 * nc, h, w)


# speedup vs baseline: 3.1743x; 1.0087x over previous
import jax
import jax.numpy as jnp
from jax.experimental import pallas as pl
from jax.experimental.pallas import tpu as pltpu


def _pos_kernel(col_ref, row_ref, o_hbm, scratch, sem):
    nc = col_ref.shape[1]
    w = col_ref.shape[0]
    h = row_ref.shape[0]
    col_t = col_ref[...].T
    row_t = row_ref[...].T
    scratch[:nc] = jnp.broadcast_to(col_t[:, None, :], (nc, h, w)).reshape(nc, h * w)
    scratch[nc:] = jnp.broadcast_to(row_t[:, :, None], (nc, h, w)).reshape(nc, h * w)
    b_total = o_hbm.shape[0]
    for b in range(b_total):
        pltpu.make_async_copy(scratch, o_hbm.at[b], sem).start()
    for b in range(b_total):
        pltpu.make_async_copy(scratch, o_hbm.at[b], sem).wait()


def kernel(x, row_w, col_w):
    b = x.shape[0]
    h, w = x.shape[-2], x.shape[-1]
    nc = row_w.shape[1]
    out = pl.pallas_call(
        _pos_kernel,
        in_specs=[
            pl.BlockSpec(memory_space=pltpu.MemorySpace.VMEM),
            pl.BlockSpec(memory_space=pltpu.MemorySpace.VMEM),
        ],
        out_specs=pl.BlockSpec(memory_space=pl.ANY),
        out_shape=jax.ShapeDtypeStruct((b, 2 * nc, h * w), jnp.float32),
        scratch_shapes=[
            pltpu.VMEM((2 * nc, h * w), jnp.float32),
            pltpu.SemaphoreType.DMA,
        ],
    )(col_w, row_w)
    return out.reshape(b, 2 * nc, h, w)
